# baseline (speedup 1.0000x reference)
import jax
import jax.numpy as jnp
from jax import lax
from jax.experimental import pallas as pl
from jax.experimental.pallas import tpu as pltpu
from jax.experimental.pallas import tpu_sc as plsc

_B, _F, _D = 16384, 26, 64
_N = _B * _F
_OD = 2 * _D
_NC, _NS, _L = 2, 16, 16
_NW = _NC * _NS
_PER_W = _N // _NW
_CH = 256
_NSUPER = _PER_W // (2 * _CH)


def _popcount16(x):
    v = x - ((x >> 1) & 0x5555)
    v = (v & 0x3333) + ((v >> 2) & 0x3333)
    v = (v + (v >> 4)) & 0x0F0F
    return (v + (v >> 8)) & 0x1F


def _div3(y):
    return lax.shift_right_logical(y * 43691, 17)


def _sc_body(x_hbm, e2_hbm, e3_hbm, out_hbm,
             xv, e2v, e3v, pf, c1f, c2f, obuf_a, obuf_b, sem_a, sem_b):
    wid = lax.axis_index("s") * _NC + lax.axis_index("c")
    base = wid * _PER_W
    pltpu.sync_copy(x_hbm.at[pl.ds(base, _PER_W)], xv)
    pltpu.sync_copy(e2_hbm, e2v)
    pltpu.sync_copy(e3_hbm, e3v)

    a2, d2, a3, d31, d32 = [], [], [], [], []
    for j in range(4):
        r0 = e2v[pl.ds(j * _L, _L)]
        r1 = e2v[pl.ds(_D + j * _L, _L)]
        a2.append(16.0 * r0)
        d2.append(r1 - r0)
        s0 = e3v[pl.ds(j * _L, _L)]
        s1 = e3v[pl.ds(_D + j * _L, _L)]
        s2 = e3v[pl.ds(2 * _D + j * _L, _L)]
        a3.append(16.0 * s0)
        d31.append(s1 - s0)
        d32.append(s2 - s0)

    def counts_for_chunk(chunk_off):
        def cbody(v, _):
            x = xv[pl.ds(chunk_off + v * _L, _L)]
            p = _popcount16(x)
            s = x - x
            n2 = x - x
            y = x
            for _i in range(11):
                q = _div3(y)
                d = y - (q + (q << 1))
                s = s + d
                n2 = n2 + (d >> 1)
                y = q
            n1 = s - (n2 << 1)
            o = v * _L
            pf[pl.ds(o, _L)] = p.astype(jnp.float32)
            c1f[pl.ds(o, _L)] = n1.astype(jnp.float32)
            c2f[pl.ds(o, _L)] = n2.astype(jnp.float32)
            return 0
        lax.fori_loop(0, _CH // _L, cbody, 0)

    def emit_chunk(obuf):
        def ebody(g, _):
            o = g * _L
            pv = pf[pl.ds(o, _L)]
            t1v = c1f[pl.ds(o, _L)]
            t2v = c2f[pl.ds(o, _L)]
            for u in range(_L):
                p = pv[u]
                t1 = t1v[u]
                t2 = t2v[u]
                eb = (o + u) * _OD
                for j in range(4):
                    obuf[pl.ds(eb + j * _L, _L)] = a2[j] + p * d2[j]
                for j in range(4):
                    obuf[pl.ds(eb + _D + j * _L, _L)] = (
                        a3[j] + t1 * d31[j] + t2 * d32[j])
            return 0
        lax.fori_loop(0, _CH // _L, ebody, 0)

    def do_chunk(s, which, obuf, sem):
        c0 = (2 * s + which) * _CH

        @pl.when(s > 0)
        def _wait_prev():
            pltpu.make_async_copy(
                obuf, out_hbm.at[pl.ds(0, _CH * _OD)], sem).wait()

        counts_for_chunk(c0)
        emit_chunk(obuf)
        pltpu.make_async_copy(
            obuf, out_hbm.at[pl.ds((base + c0) * _OD, _CH * _OD)], sem
        ).start()

    def sbody(s, _):
        do_chunk(s, 0, obuf_a, sem_a)
        do_chunk(s, 1, obuf_b, sem_b)
        return 0

    lax.fori_loop(0, _NSUPER, sbody, 0)
    pltpu.make_async_copy(obuf_a, out_hbm.at[pl.ds(0, _CH * _OD)], sem_a).wait()
    pltpu.make_async_copy(obuf_b, out_hbm.at[pl.ds(0, _CH * _OD)], sem_b).wait()


@jax.jit
def kernel(input, emb2, emb3):
    run = pl.kernel(
        _sc_body,
        out_type=jax.ShapeDtypeStruct((_N * _OD,), jnp.float32),
        mesh=plsc.VectorSubcoreMesh(core_axis_name="c", subcore_axis_name="s"),
        scratch_types=[
            pltpu.VMEM((_PER_W,), jnp.int32),
            pltpu.VMEM((_OD,), jnp.float32),
            pltpu.VMEM((3 * _D,), jnp.float32),
            pltpu.VMEM((_CH,), jnp.float32),
            pltpu.VMEM((_CH,), jnp.float32),
            pltpu.VMEM((_CH,), jnp.float32),
            pltpu.VMEM((_CH * _OD,), jnp.float32),
            pltpu.VMEM((_CH * _OD,), jnp.float32),
            pltpu.SemaphoreType.DMA,
            pltpu.SemaphoreType.DMA,
        ],
    )
    out = run(input.reshape(_N), emb2.reshape(_OD), emb3.reshape(3 * _D))
    return out.reshape(_B, _F, _OD)
